# Initial kernel scaffold; baseline (speedup 1.0000x reference)
import jax
import jax.numpy as jnp
from jax.experimental import pallas as pl


def kernel(inputs, embeddings):
    raise NotImplementedError("write your pallas kernel here")


# speedup vs baseline: 5.0834x; 5.0834x over previous
import functools

import jax
import jax.numpy as jnp
from jax import lax
from jax.experimental import pallas as pl
from jax.experimental.pallas import tpu as pltpu
from jax.experimental.pallas import tpu_sc as plsc

V = 8
N = 16384
D = 32
K = 512

N_BLK = 2048
NB = N // N_BLK

NW = 32
ROWS_PER_W = (V * N) // NW
CH = 128
NCH = ROWS_PER_W // CH


def _dist_argmin_kernel(x_ref, w_ref, idx_ref, part_ref):
    v = pl.program_id(0)
    nb = pl.program_id(1)
    x = x_ref[0]
    w = w_ref[0]
    x2 = jnp.sum(x * x, axis=1, keepdims=True)
    w2 = jnp.sum(w * w, axis=0, keepdims=True)
    mm = jnp.dot(x, w, preferred_element_type=jnp.float32)
    dist = x2 - 2.0 * mm + w2
    idx = jnp.argmin(dist, axis=1).astype(jnp.int32)
    idx_ref[0, 0, 0] = idx + v * K
    part_ref[v, nb] = jnp.sum(jnp.min(dist, axis=1))


_dist_argmin = pl.pallas_call(
    _dist_argmin_kernel,
    grid=(V, NB),
    in_specs=[
        pl.BlockSpec((1, N_BLK, D), lambda v, nb: (v, nb, 0)),
        pl.BlockSpec((1, D, K), lambda v, nb: (v, 0, 0)),
    ],
    out_specs=[
        pl.BlockSpec((1, 1, 1, N_BLK), lambda v, nb: (v, nb, 0, 0)),
        pl.BlockSpec((V, NB), lambda v, nb: (0, 0), memory_space=pltpu.SMEM),
    ],
    out_shape=[
        jax.ShapeDtypeStruct((V, NB, 1, N_BLK), jnp.int32),
        jax.ShapeDtypeStruct((V, NB), jnp.float32),
    ],
)


@functools.partial(
    pl.kernel,
    out_type=jax.ShapeDtypeStruct((V * N, D), jnp.float32),
    mesh=plsc.VectorSubcoreMesh(core_axis_name="c", subcore_axis_name="s"),
    scratch_types=[
        pltpu.VMEM((CH,), jnp.int32),
        pltpu.VMEM((CH, D), jnp.float32),
        pltpu.SemaphoreType.DMA,
    ],
    compiler_params=pltpu.CompilerParams(use_tc_tiling_on_sc=False),
)
def _sc_gather(table_hbm, idx_hbm, out_hbm, idx_v, rows_v, sem):
    wid = lax.axis_index("s") * 2 + lax.axis_index("c")
    base = wid * ROWS_PER_W

    def body(i, carry):
        off = base + i * CH
        pltpu.sync_copy(idx_hbm.at[pl.ds(off, CH)], idx_v)
        pltpu.async_copy(table_hbm.at[idx_v], rows_v, sem).wait()
        pltpu.sync_copy(rows_v, out_hbm.at[pl.ds(off, CH)])
        return carry

    lax.fori_loop(0, NCH, body, 0)


def kernel(inputs, embeddings):
    idx4, parts = _dist_argmin(inputs, embeddings)
    idx_flat = idx4.reshape(V * N)
    table = jnp.transpose(embeddings, (0, 2, 1)).reshape(V * K, D)
    out_flat = _sc_gather(table, idx_flat)
    output = out_flat.reshape(V, N, D)
    loss = 1.25 * (jnp.sum(parts) / float(V * N * D))
    return output, loss


# speedup vs baseline: 8.3146x; 1.6356x over previous
import functools

import jax
import jax.numpy as jnp
from jax import lax
from jax.experimental import pallas as pl
from jax.experimental.pallas import tpu as pltpu
from jax.experimental.pallas import tpu_sc as plsc

V = 8
N = 16384
D = 32
K = 512

N_BLK = 2048
NB = N // N_BLK

NW = 32
ROWS_PER_W = (V * N) // NW
CH = 128
NCH = ROWS_PER_W // CH


def _dist_argmin_kernel(xt_ref, wt_ref, w2_ref, idx_ref, part_ref):
    v = pl.program_id(0)
    nb = pl.program_id(1)
    xt = xt_ref[0]
    wt = wt_ref[0]
    w2 = w2_ref[0]
    x2 = jnp.sum(xt * xt, axis=0, keepdims=True)
    mmt = jnp.dot(wt, xt, preferred_element_type=jnp.float32)
    dist = (x2 - 2.0 * mmt) + w2
    idx = jnp.argmin(dist, axis=0).astype(jnp.int32)
    idx_ref[0, 0, 0] = idx + v * K
    part_ref[v, nb] = jnp.sum(jnp.min(dist, axis=0))


_dist_argmin = pl.pallas_call(
    _dist_argmin_kernel,
    grid=(V, NB),
    in_specs=[
        pl.BlockSpec((1, D, N_BLK), lambda v, nb: (v, 0, nb)),
        pl.BlockSpec((1, K, D), lambda v, nb: (v, 0, 0)),
        pl.BlockSpec((1, K, 1), lambda v, nb: (v, 0, 0)),
    ],
    out_specs=[
        pl.BlockSpec((1, 1, 1, N_BLK), lambda v, nb: (v, nb, 0, 0)),
        pl.BlockSpec((V, NB), lambda v, nb: (0, 0), memory_space=pltpu.SMEM),
    ],
    out_shape=[
        jax.ShapeDtypeStruct((V, NB, 1, N_BLK), jnp.int32),
        jax.ShapeDtypeStruct((V, NB), jnp.float32),
    ],
)


@functools.partial(
    pl.kernel,
    out_type=jax.ShapeDtypeStruct((V * N, D), jnp.float32),
    mesh=plsc.VectorSubcoreMesh(core_axis_name="c", subcore_axis_name="s"),
    scratch_types=[
        pltpu.VMEM((CH,), jnp.int32),
        pltpu.VMEM((CH, D), jnp.float32),
        pltpu.SemaphoreType.DMA,
    ],
    compiler_params=pltpu.CompilerParams(use_tc_tiling_on_sc=False),
)
def _sc_gather(table_hbm, idx_hbm, out_hbm, idx_v, rows_v, sem):
    wid = lax.axis_index("s") * 2 + lax.axis_index("c")
    base = wid * ROWS_PER_W

    def body(i, carry):
        off = base + i * CH
        pltpu.sync_copy(idx_hbm.at[pl.ds(off, CH)], idx_v)
        pltpu.async_copy(table_hbm.at[idx_v], rows_v, sem).wait()
        pltpu.sync_copy(rows_v, out_hbm.at[pl.ds(off, CH)])
        return carry

    lax.fori_loop(0, NCH, body, 0)


def kernel(inputs, embeddings):
    xt = jnp.transpose(inputs, (0, 2, 1))
    wt = jnp.transpose(embeddings, (0, 2, 1))
    w2 = jnp.sum(embeddings ** 2, axis=1)[:, :, None]
    idx4, parts = _dist_argmin(xt, wt, w2)
    idx_flat = idx4.reshape(V * N)
    table = wt.reshape(V * K, D)
    out_flat = _sc_gather(table, idx_flat)
    output = out_flat.reshape(V, N, D)
    loss = 1.25 * (jnp.sum(parts) / float(V * N * D))
    return output, loss


# speedup vs baseline: 8.9854x; 1.0807x over previous
import functools

import jax
import jax.numpy as jnp
from jax import lax
from jax.experimental import pallas as pl
from jax.experimental.pallas import tpu as pltpu
from jax.experimental.pallas import tpu_sc as plsc

V = 8
N = 16384
D = 32
K = 512

N_BLK = 2048
NB = N // N_BLK

NW = 32
ROWS_PER_W = (V * N) // NW
CH = 128
NCH = ROWS_PER_W // CH


CH_K = 8


def _dist_argmin_kernel(xt_ref, wtn_ref, w2_ref, idx_ref, part_ref):
    v = pl.program_id(0)
    nb = pl.program_id(1)
    xt = xt_ref[0]
    wtn = wtn_ref[0]
    w2 = w2_ref[0]
    x2 = jnp.sum(xt * xt, axis=0, keepdims=True)
    mmt = jnp.dot(wtn, xt, preferred_element_type=jnp.float32)
    val = jnp.full((CH_K, N_BLK), jnp.inf, jnp.float32)
    ich = jnp.zeros((CH_K, N_BLK), jnp.int32)
    for i in range(K // CH_K):
        sl = slice(i * CH_K, (i + 1) * CH_K)
        d = (x2 + mmt[sl, :]) + w2[sl, :]
        c = d < val
        val = jnp.where(c, d, val)
        ich = jnp.where(c, i, ich)
    sub = jax.lax.broadcasted_iota(jnp.int32, (CH_K, N_BLK), 0)
    kidx = ich * CH_K + sub
    m = jnp.min(val, axis=0, keepdims=True)
    kbest = jnp.min(jnp.where(val == m, kidx, K), axis=0)
    idx_ref[...] = kbest + v * K
    part_ref[v, nb] = jnp.sum(m)


_dist_argmin = pl.pallas_call(
    _dist_argmin_kernel,
    grid=(V, NB),
    in_specs=[
        pl.BlockSpec((1, D, N_BLK), lambda v, nb: (v, 0, nb)),
        pl.BlockSpec((1, K, D), lambda v, nb: (v, 0, 0)),
        pl.BlockSpec((1, K, 1), lambda v, nb: (v, 0, 0)),
    ],
    out_specs=[
        pl.BlockSpec((N_BLK,), lambda v, nb: (v * NB + nb,)),
        pl.BlockSpec((V, NB), lambda v, nb: (0, 0), memory_space=pltpu.SMEM),
    ],
    out_shape=[
        jax.ShapeDtypeStruct((V * N,), jnp.int32),
        jax.ShapeDtypeStruct((V, NB), jnp.float32),
    ],
)


@functools.partial(
    pl.kernel,
    out_type=jax.ShapeDtypeStruct((V * N, D), jnp.float32),
    mesh=plsc.VectorSubcoreMesh(core_axis_name="c", subcore_axis_name="s"),
    scratch_types=[
        pltpu.VMEM((CH,), jnp.int32),
        pltpu.VMEM((CH, D), jnp.float32),
        pltpu.SemaphoreType.DMA,
    ],
    compiler_params=pltpu.CompilerParams(use_tc_tiling_on_sc=False),
)
def _sc_gather(table_hbm, idx_hbm, out_hbm, idx_v, rows_v, sem):
    wid = lax.axis_index("s") * 2 + lax.axis_index("c")
    base = wid * ROWS_PER_W

    def body(i, carry):
        off = base + i * CH
        pltpu.sync_copy(idx_hbm.at[pl.ds(off, CH)], idx_v)
        pltpu.async_copy(table_hbm.at[idx_v], rows_v, sem).wait()
        pltpu.sync_copy(rows_v, out_hbm.at[pl.ds(off, CH)])
        return carry

    lax.fori_loop(0, NCH, body, 0)


def kernel(inputs, embeddings):
    xt = jnp.transpose(inputs, (0, 2, 1))
    wt = jnp.transpose(embeddings, (0, 2, 1))
    w2 = jnp.sum(embeddings ** 2, axis=1)[:, :, None]
    idx_flat, parts = _dist_argmin(xt, -2.0 * wt, w2)
    table = wt.reshape(V * K, D)
    out_flat = _sc_gather(table, idx_flat)
    output = out_flat.reshape(V, N, D)
    loss = 1.25 * (jnp.sum(parts) / float(V * N * D))
    return output, loss
